# Initial kernel scaffold; baseline (speedup 1.0000x reference)
import jax
import jax.numpy as jnp
from jax.experimental import pallas as pl


def kernel(x, edge_index, batch, mask, eps, W_in, b_in, Wm, bm, Wr, br, Wu, bu, Wt, bt, W_mu, b_mu, W_lv, b_lv):
    raise NotImplementedError("write your pallas kernel here")


# speedup vs baseline: 4.6953x; 4.6953x over previous
import functools

import jax
import jax.numpy as jnp
from jax import lax
from jax.experimental import pallas as pl
from jax.experimental.pallas import tpu as pltpu
from jax.experimental.pallas import tpu_sc as plsc

P = 10240
NB = 1024
NCORES = 2
NSUB = 16
NW = NCORES * NSUB
EB = 128

_F32 = jnp.float32


def _dot(a, b):
    return jnp.dot(a, b, preferred_element_type=_F32)


def _pre_body(x_ref, m_ref, win_ref, bin_ref, wm_ref, bm_ref,
              st_ref, msg_ref):
    xd = x_ref[...] * m_ref[...]
    st = jnp.maximum(_dot(xd, win_ref[...]) + bin_ref[...], 0.0)
    st_ref[...] = st
    msg_ref[...] = jnp.maximum(_dot(st, wm_ref[...]) + bm_ref[...], 0.0)


def _gru_core(st, agg, wr_ref, br_ref, wu_ref, bu_ref, wt_ref, bt_ref):
    s = st.shape[1]
    wr = wr_ref[...]
    wu = wu_ref[...]
    wt = wt_ref[...]
    r = jax.nn.sigmoid(_dot(st, wr[:s]) + _dot(agg, wr[s:]) + br_ref[...])
    u = jax.nn.sigmoid(_dot(st, wu[:s]) + _dot(agg, wu[s:]) + bu_ref[...])
    cand = jnp.tanh(_dot(st * r, wt[:s]) + _dot(agg, wt[s:]) + bt_ref[...])
    return st * (1.0 - u) + cand * u


def _gru_body(st_ref, parts_ref, wr_ref, br_ref, wu_ref, bu_ref,
              wt_ref, bt_ref, wm_ref, bm_ref, out_st_ref, out_msg_ref):
    s = st_ref.shape[1]
    agg = parts_ref[0, :, :s] + parts_ref[1, :, :s]
    ns = _gru_core(st_ref[...], agg, wr_ref, br_ref, wu_ref, bu_ref,
                   wt_ref, bt_ref)
    out_st_ref[...] = ns
    out_msg_ref[...] = jnp.maximum(_dot(ns, wm_ref[...]) + bm_ref[...], 0.0)


def _head_body(st_ref, parts_ref, wr_ref, br_ref, wu_ref, bu_ref,
               wt_ref, bt_ref, wmu_ref, bmu_ref, wlv_ref, blv_ref, eps_ref,
               z_ref, mean_ref, lv_ref):
    s = st_ref.shape[1]
    agg = parts_ref[0, :, :s] + parts_ref[1, :, :s]
    ns = _gru_core(st_ref[...], agg, wr_ref, br_ref, wu_ref, bu_ref,
                   wt_ref, bt_ref)
    mean = _dot(ns, wmu_ref[...]) + bmu_ref[...]
    lv = _dot(ns, wlv_ref[...]) + blv_ref[...]
    z_ref[...] = mean + eps_ref[...] * jnp.exp(0.5 * lv)
    mean_ref[...] = mean
    lv_ref[...] = lv


def _zdc_body(zr_ref, zct_ref, out_ref):
    out_ref[...] = jax.nn.sigmoid(_dot(zr_ref[...], zct_ref[...]))


def _make_sc_scatter(chunks, s_dim):
    mesh = plsc.VectorSubcoreMesh(core_axis_name="c", subcore_axis_name="s")
    rows_per = P // NSUB
    nz = rows_per // EB
    nlane = 16

    @functools.partial(
        pl.kernel,
        out_type=jax.ShapeDtypeStruct((NCORES, P, s_dim), _F32),
        mesh=mesh,
        compiler_params=pltpu.CompilerParams(use_tc_tiling_on_sc=False),
        scratch_types=[
            pltpu.VMEM((chunks, EB), jnp.int32),
            pltpu.VMEM((chunks, EB), jnp.int32),
            pltpu.VMEM((chunks, EB), jnp.int32),
            pltpu.VMEM((EB, s_dim), _F32),
            pltpu.VMEM((EB, s_dim), _F32),
            pltpu.VMEM((EB, s_dim), _F32),
            pltpu.VMEM_SHARED((P, s_dim), _F32),
            pltpu.SemaphoreType.DMA,
            pltpu.SemaphoreType.DMA,
        ],
    )
    def sc_scatter(msg_hbm, pk_hbm, out_hbm,
                   pk_v, src_v, dst_v, rows0, rows1, zrows, acc_sh,
                   sem0, sem1):
        c = lax.axis_index("c")
        sub = lax.axis_index("s")
        base = sub * rows_per
        wid = c * NSUB + sub
        pltpu.sync_copy(pk_hbm.at[wid], pk_v)
        zv = jnp.zeros((nlane,), _F32)
        for i in range(EB):
            for k in range(s_dim // nlane):
                zrows[i, pl.ds(k * nlane, nlane)] = zv
        for t in range(nz):
            pltpu.sync_copy(zrows, acc_sh.at[pl.ds(base + t * EB, EB)])
        for j in range(chunks):
            for k in range(EB // nlane):
                sl = pl.ds(k * nlane, nlane)
                v = pk_v[j, sl]
                src_v[j, sl] = v & 0x3FFF
                dst_v[j, sl] = lax.shift_right_logical(v, 14)
        plsc.subcore_barrier()
        bufs = (rows0, rows1)
        sems = (sem0, sem1)
        cps = [None, None]
        cps[0] = pltpu.async_copy(msg_hbm.at[src_v.at[0]], rows0, sem0)
        for j in range(chunks):
            b = j & 1
            nb = b ^ 1
            if j + 1 < chunks:
                cps[nb] = pltpu.async_copy(
                    msg_hbm.at[src_v.at[j + 1]], bufs[nb], sems[nb])
            cps[b].wait()
            pltpu.sync_copy(bufs[b], acc_sh.at[dst_v.at[j]], add=True)
        plsc.subcore_barrier()
        for t in range(nz):
            sl = pl.ds(base + t * EB, EB)
            pltpu.sync_copy(acc_sh.at[sl], zrows)
            pltpu.sync_copy(zrows, out_hbm.at[c, sl])

    return sc_scatter


def kernel(x, edge_index, batch, mask, eps, W_in, b_in, Wm, bm, Wr, br,
           Wu, bu, Wt, bt, W_mu, b_mu, W_lv, b_lv):
    n, f = x.shape
    s = W_in.shape[1]
    rounds = Wm.shape[0]
    padn = P - n

    xp = jnp.pad(x, ((0, padn), (0, 0)))
    maskp = jnp.pad(mask, ((0, padn), (0, 0)))
    epsp = jnp.pad(eps, ((0, padn), (0, 0)))

    src = edge_index[0]
    dst = edge_index[1]
    e = src.shape[0]
    chunks = -(-e // (NW * EB))
    ep = NW * chunks * EB
    pad_e = ep - e
    srcp = jnp.pad(src, (0, pad_e))
    dst_fill = n + (jnp.arange(pad_e, dtype=jnp.int32) % (P - n))
    dstp = jnp.concatenate([dst, dst_fill])
    packed = ((dstp << 14) | srcp).reshape(NW, chunks, EB)

    b_in2 = b_in.reshape(1, s)
    bm2 = bm.reshape(rounds, 1, s)
    br2 = br.reshape(rounds, 1, s)
    bu2 = bu.reshape(rounds, 1, s)
    bt2 = bt.reshape(rounds, 1, s)
    b_mu2 = b_mu.reshape(1, s)
    b_lv2 = b_lv.reshape(1, s)

    grid = (P // NB,)
    row_spec = lambda c: pl.BlockSpec((NB, c), lambda i: (i, 0))
    parts_spec = pl.BlockSpec((NCORES, NB, s), lambda i: (0, i, 0))
    full = lambda shp: pl.BlockSpec(shp, lambda i: tuple(0 for _ in shp))

    state, msg = pl.pallas_call(
        _pre_body,
        grid=grid,
        in_specs=[row_spec(f), row_spec(f), full((f, s)), full((1, s)),
                  full((s, s)), full((1, s))],
        out_specs=[row_spec(s), row_spec(s)],
        out_shape=[jax.ShapeDtypeStruct((P, s), _F32)] * 2,
    )(xp, maskp, W_in, b_in2, Wm[0], bm2[0])

    sc_scatter = _make_sc_scatter(chunks, s)

    gru_call = pl.pallas_call(
        _gru_body,
        grid=grid,
        in_specs=[row_spec(s), parts_spec,
                  full((2 * s, s)), full((1, s)),
                  full((2 * s, s)), full((1, s)),
                  full((2 * s, s)), full((1, s)),
                  full((s, s)), full((1, s))],
        out_specs=[row_spec(s), row_spec(s)],
        out_shape=[jax.ShapeDtypeStruct((P, s), _F32)] * 2,
    )

    head_call = pl.pallas_call(
        _head_body,
        grid=grid,
        in_specs=[row_spec(s), parts_spec,
                  full((2 * s, s)), full((1, s)),
                  full((2 * s, s)), full((1, s)),
                  full((2 * s, s)), full((1, s)),
                  full((s, s)), full((1, s)),
                  full((s, s)), full((1, s)),
                  row_spec(s)],
        out_specs=[row_spec(s)] * 3,
        out_shape=[jax.ShapeDtypeStruct((P, s), _F32)] * 3,
    )

    for r in range(rounds):
        parts = sc_scatter(msg, packed)
        if r + 1 < rounds:
            state, msg = gru_call(
                state, parts, Wr[r], br2[r], Wu[r], bu2[r], Wt[r], bt2[r],
                Wm[r + 1], bm2[r + 1])
        else:
            z_p, mean_p, lv_p = head_call(
                state, parts, Wr[r], br2[r], Wu[r], bu2[r], Wt[r], bt2[r],
                W_mu, b_mu2, W_lv, b_lv2, epsp)

    z = z_p[:n]
    zt = z_p.T

    bm_z, bn_z = 1024, 1024
    gm = -(-n // bm_z)
    gn = -(-n // bn_z)
    z_dc = pl.pallas_call(
        _zdc_body,
        grid=(gm, gn),
        in_specs=[pl.BlockSpec((bm_z, s), lambda i, j: (i, 0)),
                  pl.BlockSpec((s, bn_z), lambda i, j: (0, j))],
        out_specs=pl.BlockSpec((bm_z, bn_z), lambda i, j: (i, j)),
        out_shape=jax.ShapeDtypeStruct((n, n), _F32),
    )(z_p, zt)

    return (z, z_dc, mean_p[:n], lv_p[:n])


# speedup vs baseline: 4.7054x; 1.0022x over previous
import functools

import jax
import jax.numpy as jnp
from jax import lax
from jax.experimental import pallas as pl
from jax.experimental.pallas import tpu as pltpu
from jax.experimental.pallas import tpu_sc as plsc

P = 10240
NB = 1024
NCORES = 2
NSUB = 16
NW = NCORES * NSUB
EB = 128

_F32 = jnp.float32


def _dot(a, b):
    return jnp.dot(a, b, preferred_element_type=_F32)


def _pre_body(x_ref, m_ref, win_ref, bin_ref, wm_ref, bm_ref,
              st_ref, msg_ref):
    xd = x_ref[...] * m_ref[...]
    st = jnp.maximum(_dot(xd, win_ref[...]) + bin_ref[...], 0.0)
    st_ref[...] = st
    msg_ref[...] = jnp.maximum(_dot(st, wm_ref[...]) + bm_ref[...], 0.0)


def _gru_core(st, agg, wr_ref, br_ref, wu_ref, bu_ref, wt_ref, bt_ref):
    s = st.shape[1]
    wr = wr_ref[...]
    wu = wu_ref[...]
    wt = wt_ref[...]
    r = jax.nn.sigmoid(_dot(st, wr[:s]) + _dot(agg, wr[s:]) + br_ref[...])
    u = jax.nn.sigmoid(_dot(st, wu[:s]) + _dot(agg, wu[s:]) + bu_ref[...])
    cand = jnp.tanh(_dot(st * r, wt[:s]) + _dot(agg, wt[s:]) + bt_ref[...])
    return st * (1.0 - u) + cand * u


def _gru_body(st_ref, parts_ref, wr_ref, br_ref, wu_ref, bu_ref,
              wt_ref, bt_ref, wm_ref, bm_ref, out_st_ref, out_msg_ref):
    s = st_ref.shape[1]
    agg = parts_ref[0, :, :s] + parts_ref[1, :, :s]
    ns = _gru_core(st_ref[...], agg, wr_ref, br_ref, wu_ref, bu_ref,
                   wt_ref, bt_ref)
    out_st_ref[...] = ns
    out_msg_ref[...] = jnp.maximum(_dot(ns, wm_ref[...]) + bm_ref[...], 0.0)


def _head_body(st_ref, parts_ref, wr_ref, br_ref, wu_ref, bu_ref,
               wt_ref, bt_ref, wmu_ref, bmu_ref, wlv_ref, blv_ref, eps_ref,
               z_ref, mean_ref, lv_ref):
    s = st_ref.shape[1]
    agg = parts_ref[0, :, :s] + parts_ref[1, :, :s]
    ns = _gru_core(st_ref[...], agg, wr_ref, br_ref, wu_ref, bu_ref,
                   wt_ref, bt_ref)
    mean = _dot(ns, wmu_ref[...]) + bmu_ref[...]
    lv = _dot(ns, wlv_ref[...]) + blv_ref[...]
    z_ref[...] = mean + eps_ref[...] * jnp.exp(0.5 * lv)
    mean_ref[...] = mean
    lv_ref[...] = lv


def _zdc_body(zr_ref, zct_ref, out_ref):
    out_ref[...] = jax.nn.sigmoid(_dot(zr_ref[...], zct_ref[...]))


_NBUF = 8
_PDIST = 4


def _make_sc_scatter(chunks, s_dim):
    mesh = plsc.VectorSubcoreMesh(core_axis_name="c", subcore_axis_name="s")
    rows_per = P // NSUB
    nz = rows_per // EB
    nlane = 16

    @functools.partial(
        pl.kernel,
        out_type=jax.ShapeDtypeStruct((NCORES, P, s_dim), _F32),
        mesh=mesh,
        compiler_params=pltpu.CompilerParams(use_tc_tiling_on_sc=False),
        scratch_types=[
            pltpu.VMEM((chunks, EB), jnp.int32),
            pltpu.VMEM((chunks, EB), jnp.int32),
            pltpu.VMEM((chunks, EB), jnp.int32),
            [pltpu.VMEM((EB, s_dim), _F32)] * _NBUF,
            pltpu.VMEM((EB, s_dim), _F32),
            pltpu.VMEM_SHARED((P, s_dim), _F32),
            [pltpu.SemaphoreType.DMA] * _NBUF,
            pltpu.SemaphoreType.DMA,
        ],
    )
    def sc_scatter(msg_hbm, pk_hbm, out_hbm,
                   pk_v, src_v, dst_v, bufs, zrows, acc_sh, gsems, ssem):
        c = lax.axis_index("c")
        sub = lax.axis_index("s")
        base = sub * rows_per
        wid = c * NSUB + sub
        pltpu.sync_copy(pk_hbm.at[wid], pk_v)
        for j in range(chunks):
            for k in range(EB // nlane):
                sl = pl.ds(k * nlane, nlane)
                v = pk_v[j, sl]
                src_v[j, sl] = v & 0x3FFF
                dst_v[j, sl] = lax.shift_right_logical(v, 14)
        gcp = [None] * _NBUF
        for g in range(min(_PDIST, chunks)):
            gcp[g] = pltpu.async_copy(
                msg_hbm.at[src_v.at[g]], bufs[g], gsems[g])
        zv = jnp.zeros((nlane,), _F32)
        for i in range(EB):
            for k in range(s_dim // nlane):
                zrows[i, pl.ds(k * nlane, nlane)] = zv
        for t in range(nz):
            pltpu.sync_copy(zrows, acc_sh.at[pl.ds(base + t * EB, EB)])
        plsc.subcore_barrier()
        scp = None
        for j in range(chunks):
            b = j % _NBUF
            gcp[b].wait()
            if scp is not None:
                scp.wait()
            scp = pltpu.async_copy(
                bufs[b], acc_sh.at[dst_v.at[j]], ssem, add=True)
            g = j + _PDIST
            if g < chunks:
                gb = g % _NBUF
                gcp[gb] = pltpu.async_copy(
                    msg_hbm.at[src_v.at[g]], bufs[gb], gsems[gb])
        if scp is not None:
            scp.wait()
        plsc.subcore_barrier()
        for t in range(nz):
            sl = pl.ds(base + t * EB, EB)
            pltpu.sync_copy(acc_sh.at[sl], zrows)
            pltpu.sync_copy(zrows, out_hbm.at[c, sl])

    return sc_scatter


def kernel(x, edge_index, batch, mask, eps, W_in, b_in, Wm, bm, Wr, br,
           Wu, bu, Wt, bt, W_mu, b_mu, W_lv, b_lv):
    n, f = x.shape
    s = W_in.shape[1]
    rounds = Wm.shape[0]
    padn = P - n

    xp = jnp.pad(x, ((0, padn), (0, 0)))
    maskp = jnp.pad(mask, ((0, padn), (0, 0)))
    epsp = jnp.pad(eps, ((0, padn), (0, 0)))

    src = edge_index[0]
    dst = edge_index[1]
    e = src.shape[0]
    chunks = -(-e // (NW * EB))
    ep = NW * chunks * EB
    pad_e = ep - e
    srcp = jnp.pad(src, (0, pad_e))
    dst_fill = n + (jnp.arange(pad_e, dtype=jnp.int32) % (P - n))
    dstp = jnp.concatenate([dst, dst_fill])
    packed = ((dstp << 14) | srcp).reshape(NW, chunks, EB)

    b_in2 = b_in.reshape(1, s)
    bm2 = bm.reshape(rounds, 1, s)
    br2 = br.reshape(rounds, 1, s)
    bu2 = bu.reshape(rounds, 1, s)
    bt2 = bt.reshape(rounds, 1, s)
    b_mu2 = b_mu.reshape(1, s)
    b_lv2 = b_lv.reshape(1, s)

    grid = (P // NB,)
    row_spec = lambda c: pl.BlockSpec((NB, c), lambda i: (i, 0))
    parts_spec = pl.BlockSpec((NCORES, NB, s), lambda i: (0, i, 0))
    full = lambda shp: pl.BlockSpec(shp, lambda i: tuple(0 for _ in shp))

    state, msg = pl.pallas_call(
        _pre_body,
        grid=grid,
        in_specs=[row_spec(f), row_spec(f), full((f, s)), full((1, s)),
                  full((s, s)), full((1, s))],
        out_specs=[row_spec(s), row_spec(s)],
        out_shape=[jax.ShapeDtypeStruct((P, s), _F32)] * 2,
    )(xp, maskp, W_in, b_in2, Wm[0], bm2[0])

    sc_scatter = _make_sc_scatter(chunks, s)

    gru_call = pl.pallas_call(
        _gru_body,
        grid=grid,
        in_specs=[row_spec(s), parts_spec,
                  full((2 * s, s)), full((1, s)),
                  full((2 * s, s)), full((1, s)),
                  full((2 * s, s)), full((1, s)),
                  full((s, s)), full((1, s))],
        out_specs=[row_spec(s), row_spec(s)],
        out_shape=[jax.ShapeDtypeStruct((P, s), _F32)] * 2,
    )

    head_call = pl.pallas_call(
        _head_body,
        grid=grid,
        in_specs=[row_spec(s), parts_spec,
                  full((2 * s, s)), full((1, s)),
                  full((2 * s, s)), full((1, s)),
                  full((2 * s, s)), full((1, s)),
                  full((s, s)), full((1, s)),
                  full((s, s)), full((1, s)),
                  row_spec(s)],
        out_specs=[row_spec(s)] * 3,
        out_shape=[jax.ShapeDtypeStruct((P, s), _F32)] * 3,
    )

    for r in range(rounds):
        parts = sc_scatter(msg, packed)
        if r + 1 < rounds:
            state, msg = gru_call(
                state, parts, Wr[r], br2[r], Wu[r], bu2[r], Wt[r], bt2[r],
                Wm[r + 1], bm2[r + 1])
        else:
            z_p, mean_p, lv_p = head_call(
                state, parts, Wr[r], br2[r], Wu[r], bu2[r], Wt[r], bt2[r],
                W_mu, b_mu2, W_lv, b_lv2, epsp)

    z = z_p[:n]
    zt = z_p.T

    bm_z, bn_z = 1024, 1024
    gm = -(-n // bm_z)
    gn = -(-n // bn_z)
    z_dc = pl.pallas_call(
        _zdc_body,
        grid=(gm, gn),
        in_specs=[pl.BlockSpec((bm_z, s), lambda i, j: (i, 0)),
                  pl.BlockSpec((s, bn_z), lambda i, j: (0, j))],
        out_specs=pl.BlockSpec((bm_z, bn_z), lambda i, j: (i, j)),
        out_shape=jax.ShapeDtypeStruct((n, n), _F32),
    )(z_p, zt)

    return (z, z_dc, mean_p[:n], lv_p[:n])


# speedup vs baseline: 5.0391x; 1.0709x over previous
import functools

import jax
import jax.numpy as jnp
from jax import lax
from jax.experimental import pallas as pl
from jax.experimental.pallas import tpu as pltpu
from jax.experimental.pallas import tpu_sc as plsc

P = 10240
NB = 1024
NCORES = 2
NSUB = 16
NW = NCORES * NSUB
EB = 128

_F32 = jnp.float32


def _dot(a, b):
    return jnp.dot(a, b, preferred_element_type=_F32)


def _pre_body(x_ref, m_ref, win_ref, bin_ref, wm_ref, bm_ref,
              st_ref, msg_ref):
    xd = x_ref[...] * m_ref[...]
    st = jnp.maximum(_dot(xd, win_ref[...]) + bin_ref[...], 0.0)
    st_ref[...] = st
    msg_ref[...] = jnp.maximum(_dot(st, wm_ref[...]) + bm_ref[...], 0.0)


def _gru_core(st, agg, wr_ref, br_ref, wu_ref, bu_ref, wt_ref, bt_ref):
    s = st.shape[1]
    wr = wr_ref[...]
    wu = wu_ref[...]
    wt = wt_ref[...]
    r = jax.nn.sigmoid(_dot(st, wr[:s]) + _dot(agg, wr[s:]) + br_ref[...])
    u = jax.nn.sigmoid(_dot(st, wu[:s]) + _dot(agg, wu[s:]) + bu_ref[...])
    cand = jnp.tanh(_dot(st * r, wt[:s]) + _dot(agg, wt[s:]) + bt_ref[...])
    return st * (1.0 - u) + cand * u


def _gru_body(st_ref, parts_ref, wr_ref, br_ref, wu_ref, bu_ref,
              wt_ref, bt_ref, wm_ref, bm_ref, out_st_ref, out_msg_ref):
    s = st_ref.shape[1]
    agg = parts_ref[0, :, :s] + parts_ref[1, :, :s]
    ns = _gru_core(st_ref[...], agg, wr_ref, br_ref, wu_ref, bu_ref,
                   wt_ref, bt_ref)
    out_st_ref[...] = ns
    out_msg_ref[...] = jnp.maximum(_dot(ns, wm_ref[...]) + bm_ref[...], 0.0)


def _head_body(st_ref, parts_ref, wr_ref, br_ref, wu_ref, bu_ref,
               wt_ref, bt_ref, wmu_ref, bmu_ref, wlv_ref, blv_ref, eps_ref,
               z_ref, mean_ref, lv_ref):
    s = st_ref.shape[1]
    agg = parts_ref[0, :, :s] + parts_ref[1, :, :s]
    ns = _gru_core(st_ref[...], agg, wr_ref, br_ref, wu_ref, bu_ref,
                   wt_ref, bt_ref)
    mean = _dot(ns, wmu_ref[...]) + bmu_ref[...]
    lv = _dot(ns, wlv_ref[...]) + blv_ref[...]
    z_ref[...] = mean + eps_ref[...] * jnp.exp(0.5 * lv)
    mean_ref[...] = mean
    lv_ref[...] = lv


def _zdc_body(zr_ref, zct_ref, out_ref):
    out_ref[...] = jax.nn.sigmoid(_dot(zr_ref[...], zct_ref[...]))


_NBUF = 8
_PDIST = 4


def _make_sc_scatter(chunks, s_dim):
    mesh = plsc.VectorSubcoreMesh(core_axis_name="c", subcore_axis_name="s")
    rows_per = P // NSUB
    nz = rows_per // EB
    nlane = 16

    @functools.partial(
        pl.kernel,
        out_type=jax.ShapeDtypeStruct((NCORES, P, s_dim), _F32),
        mesh=mesh,
        compiler_params=pltpu.CompilerParams(use_tc_tiling_on_sc=False),
        scratch_types=[
            pltpu.VMEM((chunks, EB), jnp.int32),
            pltpu.VMEM((chunks, EB), jnp.int32),
            pltpu.VMEM((chunks, EB), jnp.int32),
            [pltpu.VMEM((EB, s_dim), _F32)] * _NBUF,
            pltpu.VMEM((EB, s_dim), _F32),
            pltpu.VMEM_SHARED((P, s_dim), _F32),
            [pltpu.SemaphoreType.DMA] * _NBUF,
            pltpu.SemaphoreType.DMA,
        ],
    )
    def sc_scatter(msg_hbm, pk_hbm, out_hbm,
                   pk_v, src_v, dst_v, bufs, zrows, acc_sh, gsems, ssem):
        c = lax.axis_index("c")
        sub = lax.axis_index("s")
        base = sub * rows_per
        wid = c * NSUB + sub
        pltpu.sync_copy(pk_hbm.at[wid], pk_v)
        for j in range(chunks):
            for k in range(EB // nlane):
                sl = pl.ds(k * nlane, nlane)
                v = pk_v[j, sl]
                src_v[j, sl] = v & 0x3FFF
                dst_v[j, sl] = lax.shift_right_logical(v, 14)
        gcp = [None] * _NBUF
        for g in range(min(_PDIST, chunks)):
            gcp[g] = pltpu.async_copy(
                msg_hbm.at[src_v.at[g]], bufs[g], gsems[g])
        zv = jnp.zeros((nlane,), _F32)
        for i in range(EB):
            for k in range(s_dim // nlane):
                zrows[i, pl.ds(k * nlane, nlane)] = zv
        zcp = [
            pltpu.async_copy(zrows, acc_sh.at[pl.ds(base + t * EB, EB)],
                             gsems[_PDIST + t % (_NBUF - _PDIST)])
            for t in range(nz)
        ]
        for cp in zcp:
            cp.wait()
        plsc.subcore_barrier()
        scp = None
        for j in range(chunks):
            b = j % _NBUF
            gcp[b].wait()
            if scp is not None:
                scp.wait()
            scp = pltpu.async_copy(
                bufs[b], acc_sh.at[dst_v.at[j]], ssem, add=True)
            g = j + _PDIST
            if g < chunks:
                gb = g % _NBUF
                gcp[gb] = pltpu.async_copy(
                    msg_hbm.at[src_v.at[g]], bufs[gb], gsems[gb])
        if scp is not None:
            scp.wait()
        plsc.subcore_barrier()
        ocp = [None] * nz
        hcp = [None] * nz
        for t in range(nz):
            sl = pl.ds(base + t * EB, EB)
            ocp[t] = pltpu.async_copy(acc_sh.at[sl], bufs[t % _NBUF],
                                      gsems[t % _NBUF])
        for t in range(nz):
            sl = pl.ds(base + t * EB, EB)
            ocp[t].wait()
            hcp[t] = pltpu.async_copy(bufs[t % _NBUF], out_hbm.at[c, sl],
                                      ssem)
        for t in range(nz):
            hcp[t].wait()

    return sc_scatter


def kernel(x, edge_index, batch, mask, eps, W_in, b_in, Wm, bm, Wr, br,
           Wu, bu, Wt, bt, W_mu, b_mu, W_lv, b_lv):
    n, f = x.shape
    s = W_in.shape[1]
    rounds = Wm.shape[0]
    padn = P - n

    xp = jnp.pad(x, ((0, padn), (0, 0)))
    maskp = jnp.pad(mask, ((0, padn), (0, 0)))
    epsp = jnp.pad(eps, ((0, padn), (0, 0)))

    src = edge_index[0]
    dst = edge_index[1]
    e = src.shape[0]
    chunks = -(-e // (NW * EB))
    ep = NW * chunks * EB
    pad_e = ep - e
    srcp = jnp.pad(src, (0, pad_e))
    dst_fill = n + (jnp.arange(pad_e, dtype=jnp.int32) % (P - n))
    dstp = jnp.concatenate([dst, dst_fill])
    packed = ((dstp << 14) | srcp).reshape(NW, chunks, EB)

    b_in2 = b_in.reshape(1, s)
    bm2 = bm.reshape(rounds, 1, s)
    br2 = br.reshape(rounds, 1, s)
    bu2 = bu.reshape(rounds, 1, s)
    bt2 = bt.reshape(rounds, 1, s)
    b_mu2 = b_mu.reshape(1, s)
    b_lv2 = b_lv.reshape(1, s)

    grid = (P // NB,)
    row_spec = lambda c: pl.BlockSpec((NB, c), lambda i: (i, 0))
    parts_spec = pl.BlockSpec((NCORES, NB, s), lambda i: (0, i, 0))
    full = lambda shp: pl.BlockSpec(shp, lambda i: tuple(0 for _ in shp))

    state, msg = pl.pallas_call(
        _pre_body,
        grid=grid,
        in_specs=[row_spec(f), row_spec(f), full((f, s)), full((1, s)),
                  full((s, s)), full((1, s))],
        out_specs=[row_spec(s), row_spec(s)],
        out_shape=[jax.ShapeDtypeStruct((P, s), _F32)] * 2,
    )(xp, maskp, W_in, b_in2, Wm[0], bm2[0])

    sc_scatter = _make_sc_scatter(chunks, s)

    gru_call = pl.pallas_call(
        _gru_body,
        grid=grid,
        in_specs=[row_spec(s), parts_spec,
                  full((2 * s, s)), full((1, s)),
                  full((2 * s, s)), full((1, s)),
                  full((2 * s, s)), full((1, s)),
                  full((s, s)), full((1, s))],
        out_specs=[row_spec(s), row_spec(s)],
        out_shape=[jax.ShapeDtypeStruct((P, s), _F32)] * 2,
    )

    head_call = pl.pallas_call(
        _head_body,
        grid=grid,
        in_specs=[row_spec(s), parts_spec,
                  full((2 * s, s)), full((1, s)),
                  full((2 * s, s)), full((1, s)),
                  full((2 * s, s)), full((1, s)),
                  full((s, s)), full((1, s)),
                  full((s, s)), full((1, s)),
                  row_spec(s)],
        out_specs=[row_spec(s)] * 3,
        out_shape=[jax.ShapeDtypeStruct((P, s), _F32)] * 3,
    )

    for r in range(rounds):
        parts = sc_scatter(msg, packed)
        if r + 1 < rounds:
            state, msg = gru_call(
                state, parts, Wr[r], br2[r], Wu[r], bu2[r], Wt[r], bt2[r],
                Wm[r + 1], bm2[r + 1])
        else:
            z_p, mean_p, lv_p = head_call(
                state, parts, Wr[r], br2[r], Wu[r], bu2[r], Wt[r], bt2[r],
                W_mu, b_mu2, W_lv, b_lv2, epsp)

    z = z_p[:n]
    zt = z_p.T

    bm_z, bn_z = 2048, 2048
    gm = -(-n // bm_z)
    gn = -(-n // bn_z)
    z_dc = pl.pallas_call(
        _zdc_body,
        grid=(gm, gn),
        in_specs=[pl.BlockSpec((bm_z, s), lambda i, j: (i, 0)),
                  pl.BlockSpec((s, bn_z), lambda i, j: (0, j))],
        out_specs=pl.BlockSpec((bm_z, bn_z), lambda i, j: (i, j)),
        out_shape=jax.ShapeDtypeStruct((n, n), _F32),
    )(z_p, zt)

    return (z, z_dc, mean_p[:n], lv_p[:n])


# speedup vs baseline: 6.5279x; 1.2955x over previous
import functools

import jax
import jax.numpy as jnp
from jax import lax
from jax.experimental import pallas as pl
from jax.experimental.pallas import tpu as pltpu
from jax.experimental.pallas import tpu_sc as plsc

P = 10240
NB = 1024
NCORES = 2
NSUB = 16
NW = NCORES * NSUB
EB = 128

_F32 = jnp.float32


def _dot(a, b):
    return jnp.dot(a, b, preferred_element_type=_F32)


def _pre_body(x_ref, m_ref, win_ref, bin_ref, wm_ref, bm_ref,
              st_ref, msg_ref):
    xd = x_ref[...] * m_ref[...]
    st = jnp.maximum(_dot(xd, win_ref[...]) + bin_ref[...], 0.0)
    st_ref[...] = st
    msg_ref[...] = jnp.maximum(_dot(st, wm_ref[...]) + bm_ref[...], 0.0)


def _gru_core(st, agg, wr_ref, br_ref, wu_ref, bu_ref, wt_ref, bt_ref):
    s = st.shape[1]
    wr = wr_ref[...]
    wu = wu_ref[...]
    wt = wt_ref[...]
    r = jax.nn.sigmoid(_dot(st, wr[:s]) + _dot(agg, wr[s:]) + br_ref[...])
    u = jax.nn.sigmoid(_dot(st, wu[:s]) + _dot(agg, wu[s:]) + bu_ref[...])
    cand = jnp.tanh(_dot(st * r, wt[:s]) + _dot(agg, wt[s:]) + bt_ref[...])
    return st * (1.0 - u) + cand * u


def _gru_body(st_ref, parts_ref, wr_ref, br_ref, wu_ref, bu_ref,
              wt_ref, bt_ref, wm_ref, bm_ref, out_st_ref, out_msg_ref):
    s = st_ref.shape[1]
    agg = parts_ref[0, :, :s] + parts_ref[1, :, :s]
    ns = _gru_core(st_ref[...], agg, wr_ref, br_ref, wu_ref, bu_ref,
                   wt_ref, bt_ref)
    out_st_ref[...] = ns
    out_msg_ref[...] = jnp.maximum(_dot(ns, wm_ref[...]) + bm_ref[...], 0.0)


def _head_body(st_ref, parts_ref, wr_ref, br_ref, wu_ref, bu_ref,
               wt_ref, bt_ref, wmu_ref, bmu_ref, wlv_ref, blv_ref, eps_ref,
               z_ref, mean_ref, lv_ref):
    s = st_ref.shape[1]
    agg = parts_ref[0, :, :s] + parts_ref[1, :, :s]
    ns = _gru_core(st_ref[...], agg, wr_ref, br_ref, wu_ref, bu_ref,
                   wt_ref, bt_ref)
    mean = _dot(ns, wmu_ref[...]) + bmu_ref[...]
    lv = _dot(ns, wlv_ref[...]) + blv_ref[...]
    z_ref[...] = mean + eps_ref[...] * jnp.exp(0.5 * lv)
    mean_ref[...] = mean
    lv_ref[...] = lv


def _zdc_body(zr_ref, zct_ref, out_ref):
    out_ref[...] = jax.nn.sigmoid(_dot(zr_ref[...], zct_ref[...]))


_NBUF = 8
_PDIST = 4


def _make_sc_scatter(chunks, s_dim):
    mesh = plsc.VectorSubcoreMesh(core_axis_name="c", subcore_axis_name="s")
    rows_per = P // NSUB
    nz = rows_per // EB
    nlane = 16

    @functools.partial(
        pl.kernel,
        out_type=jax.ShapeDtypeStruct((NCORES, P, s_dim), _F32),
        mesh=mesh,
        compiler_params=pltpu.CompilerParams(use_tc_tiling_on_sc=False),
        scratch_types=[
            pltpu.VMEM((chunks, EB), jnp.int32),
            pltpu.VMEM((chunks, EB), jnp.int32),
            pltpu.VMEM((chunks, EB), jnp.int32),
            [pltpu.VMEM((EB, s_dim), _F32)] * _NBUF,
            pltpu.VMEM((EB, s_dim), _F32),
            pltpu.VMEM_SHARED((P, s_dim), _F32),
            pltpu.VMEM_SHARED((P, s_dim), _F32),
            [pltpu.SemaphoreType.DMA] * _NBUF,
            pltpu.SemaphoreType.DMA,
            pltpu.SemaphoreType.DMA,
        ],
    )
    def sc_scatter(msg_hbm, pk_hbm, out_hbm,
                   pk_v, src_v, dst_v, bufs, zrows, acc_sh, msg_sh,
                   gsems, ssem, msem):
        c = lax.axis_index("c")
        sub = lax.axis_index("s")
        base = sub * rows_per
        wid = c * NSUB + sub
        msl = pl.ds(base, rows_per)
        mcp = pltpu.async_copy(msg_hbm.at[msl], msg_sh.at[msl], msem)
        pltpu.sync_copy(pk_hbm.at[wid], pk_v)
        for j in range(chunks):
            for k in range(EB // nlane):
                sl = pl.ds(k * nlane, nlane)
                v = pk_v[j, sl]
                src_v[j, sl] = v & 0x3FFF
                dst_v[j, sl] = lax.shift_right_logical(v, 14)
        zv = jnp.zeros((nlane,), _F32)
        for i in range(EB):
            for k in range(s_dim // nlane):
                zrows[i, pl.ds(k * nlane, nlane)] = zv
        zcp = [
            pltpu.async_copy(zrows, acc_sh.at[pl.ds(base + t * EB, EB)],
                             gsems[_PDIST + t % (_NBUF - _PDIST)])
            for t in range(nz)
        ]
        for cp in zcp:
            cp.wait()
        mcp.wait()
        plsc.subcore_barrier()
        gcp = [None] * _NBUF
        for g in range(min(_PDIST, chunks)):
            gcp[g] = pltpu.async_copy(
                msg_sh.at[src_v.at[g]], bufs[g], gsems[g])
        scp = None
        for j in range(chunks):
            b = j % _NBUF
            gcp[b].wait()
            if scp is not None:
                scp.wait()
            scp = pltpu.async_copy(
                bufs[b], acc_sh.at[dst_v.at[j]], ssem, add=True)
            g = j + _PDIST
            if g < chunks:
                gb = g % _NBUF
                gcp[gb] = pltpu.async_copy(
                    msg_sh.at[src_v.at[g]], bufs[gb], gsems[gb])
        if scp is not None:
            scp.wait()
        plsc.subcore_barrier()
        ocp = [None] * nz
        hcp = [None] * nz
        for t in range(nz):
            sl = pl.ds(base + t * EB, EB)
            ocp[t] = pltpu.async_copy(acc_sh.at[sl], bufs[t % _NBUF],
                                      gsems[t % _NBUF])
        for t in range(nz):
            sl = pl.ds(base + t * EB, EB)
            ocp[t].wait()
            hcp[t] = pltpu.async_copy(bufs[t % _NBUF], out_hbm.at[c, sl],
                                      ssem)
        for t in range(nz):
            hcp[t].wait()

    return sc_scatter


def kernel(x, edge_index, batch, mask, eps, W_in, b_in, Wm, bm, Wr, br,
           Wu, bu, Wt, bt, W_mu, b_mu, W_lv, b_lv):
    n, f = x.shape
    s = W_in.shape[1]
    rounds = Wm.shape[0]
    padn = P - n

    xp = jnp.pad(x, ((0, padn), (0, 0)))
    maskp = jnp.pad(mask, ((0, padn), (0, 0)))
    epsp = jnp.pad(eps, ((0, padn), (0, 0)))

    src = edge_index[0]
    dst = edge_index[1]
    e = src.shape[0]
    chunks = -(-e // (NW * EB))
    ep = NW * chunks * EB
    pad_e = ep - e
    srcp = jnp.pad(src, (0, pad_e))
    dst_fill = n + (jnp.arange(pad_e, dtype=jnp.int32) % (P - n))
    dstp = jnp.concatenate([dst, dst_fill])
    packed = ((dstp << 14) | srcp).reshape(NW, chunks, EB)

    b_in2 = b_in.reshape(1, s)
    bm2 = bm.reshape(rounds, 1, s)
    br2 = br.reshape(rounds, 1, s)
    bu2 = bu.reshape(rounds, 1, s)
    bt2 = bt.reshape(rounds, 1, s)
    b_mu2 = b_mu.reshape(1, s)
    b_lv2 = b_lv.reshape(1, s)

    grid = (P // NB,)
    row_spec = lambda c: pl.BlockSpec((NB, c), lambda i: (i, 0))
    parts_spec = pl.BlockSpec((NCORES, NB, s), lambda i: (0, i, 0))
    full = lambda shp: pl.BlockSpec(shp, lambda i: tuple(0 for _ in shp))

    state, msg = pl.pallas_call(
        _pre_body,
        grid=grid,
        in_specs=[row_spec(f), row_spec(f), full((f, s)), full((1, s)),
                  full((s, s)), full((1, s))],
        out_specs=[row_spec(s), row_spec(s)],
        out_shape=[jax.ShapeDtypeStruct((P, s), _F32)] * 2,
    )(xp, maskp, W_in, b_in2, Wm[0], bm2[0])

    sc_scatter = _make_sc_scatter(chunks, s)

    gru_call = pl.pallas_call(
        _gru_body,
        grid=grid,
        in_specs=[row_spec(s), parts_spec,
                  full((2 * s, s)), full((1, s)),
                  full((2 * s, s)), full((1, s)),
                  full((2 * s, s)), full((1, s)),
                  full((s, s)), full((1, s))],
        out_specs=[row_spec(s), row_spec(s)],
        out_shape=[jax.ShapeDtypeStruct((P, s), _F32)] * 2,
    )

    head_call = pl.pallas_call(
        _head_body,
        grid=grid,
        in_specs=[row_spec(s), parts_spec,
                  full((2 * s, s)), full((1, s)),
                  full((2 * s, s)), full((1, s)),
                  full((2 * s, s)), full((1, s)),
                  full((s, s)), full((1, s)),
                  full((s, s)), full((1, s)),
                  row_spec(s)],
        out_specs=[row_spec(s)] * 3,
        out_shape=[jax.ShapeDtypeStruct((P, s), _F32)] * 3,
    )

    for r in range(rounds):
        parts = sc_scatter(msg, packed)
        if r + 1 < rounds:
            state, msg = gru_call(
                state, parts, Wr[r], br2[r], Wu[r], bu2[r], Wt[r], bt2[r],
                Wm[r + 1], bm2[r + 1])
        else:
            z_p, mean_p, lv_p = head_call(
                state, parts, Wr[r], br2[r], Wu[r], bu2[r], Wt[r], bt2[r],
                W_mu, b_mu2, W_lv, b_lv2, epsp)

    z = z_p[:n]
    zt = z_p.T

    bm_z, bn_z = 2048, 2048
    gm = -(-n // bm_z)
    gn = -(-n // bn_z)
    z_dc = pl.pallas_call(
        _zdc_body,
        grid=(gm, gn),
        in_specs=[pl.BlockSpec((bm_z, s), lambda i, j: (i, 0)),
                  pl.BlockSpec((s, bn_z), lambda i, j: (0, j))],
        out_specs=pl.BlockSpec((bm_z, bn_z), lambda i, j: (i, j)),
        out_shape=jax.ShapeDtypeStruct((n, n), _F32),
    )(z_p, zt)

    return (z, z_dc, mean_p[:n], lv_p[:n])
